# initial kernel scaffold (unmeasured)
import jax
import jax.numpy as jnp
from jax import lax
from jax.experimental import pallas as pl
from jax.experimental.pallas import tpu as pltpu


def kernel(
    x,
):
    def body(*refs):
        pass

    out_shape = jax.ShapeDtypeStruct(..., jnp.float32)
    return pl.pallas_call(body, out_shape=out_shape)(...)


# baseline (device time: 15679 ns/iter reference)
import jax
import jax.numpy as jnp
from jax import lax
from jax.experimental import pallas as pl
from jax.experimental.pallas import tpu as pltpu

N_DEV = 16
GLOBAL_ROWS = 8192


def kernel(x):
    m_per, n = x.shape
    inv = 1.0 / GLOBAL_ROWS

    def body(x_ref, out_ref, gather_ref, send_sems, recv_sems):
        my_pos = lax.axis_index("i")

        gather_ref[pl.ds(my_pos, 1), :] = (
            jnp.sum(x_ref[:, :], axis=0, keepdims=True) * inv
        )

        for q in range(N_DEV):
            @pl.when(q != my_pos)
            def _(q=q):
                rdma = pltpu.make_async_remote_copy(
                    src_ref=gather_ref.at[pl.ds(my_pos, 1), :],
                    dst_ref=gather_ref.at[pl.ds(my_pos, 1), :],
                    send_sem=send_sems.at[q],
                    recv_sem=recv_sems.at[my_pos],
                    device_id=(q,),
                    device_id_type=pl.DeviceIdType.MESH,
                )
                rdma.start()

        for s in range(N_DEV):
            @pl.when(s != my_pos)
            def _(s=s):
                recv = pltpu.make_async_remote_copy(
                    src_ref=gather_ref.at[pl.ds(s, 1), :],
                    dst_ref=gather_ref.at[pl.ds(s, 1), :],
                    send_sem=send_sems.at[s],
                    recv_sem=recv_sems.at[s],
                    device_id=(0,),
                    device_id_type=pl.DeviceIdType.MESH,
                )
                recv.wait_recv()

        out_ref[:, :] = jnp.sum(gather_ref[:, :], axis=0, keepdims=True)

        for q in range(N_DEV):
            @pl.when(q != my_pos)
            def _(q=q):
                snd = pltpu.make_async_remote_copy(
                    src_ref=gather_ref.at[pl.ds(my_pos, 1), :],
                    dst_ref=gather_ref.at[pl.ds(my_pos, 1), :],
                    send_sem=send_sems.at[q],
                    recv_sem=recv_sems.at[q],
                    device_id=(0,),
                    device_id_type=pl.DeviceIdType.MESH,
                )
                snd.wait_send()

    return pl.pallas_call(
        body,
        out_shape=jax.ShapeDtypeStruct((1, n), jnp.float32),
        in_specs=[pl.BlockSpec(memory_space=pltpu.VMEM)],
        out_specs=pl.BlockSpec(memory_space=pltpu.VMEM),
        scratch_shapes=[
            pltpu.VMEM((N_DEV, n), jnp.float32),
            pltpu.SemaphoreType.DMA((N_DEV,)),
            pltpu.SemaphoreType.DMA((N_DEV,)),
        ],
    )(x)


# device time: 9728 ns/iter; 1.6117x vs baseline; 1.6117x over previous
import jax
import jax.numpy as jnp
from jax import lax
from jax.experimental import pallas as pl
from jax.experimental.pallas import tpu as pltpu

N_DEV = 16
GLOBAL_ROWS = 8192


def kernel(x):
    m_per, n = x.shape
    inv = 1.0 / GLOBAL_ROWS

    def body(x_ref, out_ref, gather_ref, send_sems, recv_sems):
        my_pos = lax.axis_index("i")

        barrier_sem = pltpu.get_barrier_semaphore()
        for q in range(N_DEV):
            @pl.when(q != my_pos)
            def _(q=q):
                pl.semaphore_signal(
                    barrier_sem, inc=1,
                    device_id=(q,), device_id_type=pl.DeviceIdType.MESH,
                )

        gather_ref[pl.ds(my_pos, 1), :] = (
            jnp.sum(x_ref[:, :], axis=0, keepdims=True) * inv
        )

        pl.semaphore_wait(barrier_sem, N_DEV - 1)

        for q in range(N_DEV):
            @pl.when(q != my_pos)
            def _(q=q):
                rdma = pltpu.make_async_remote_copy(
                    src_ref=gather_ref.at[pl.ds(my_pos, 1), :],
                    dst_ref=gather_ref.at[pl.ds(my_pos, 1), :],
                    send_sem=send_sems.at[q],
                    recv_sem=recv_sems.at[my_pos],
                    device_id=(q,),
                    device_id_type=pl.DeviceIdType.MESH,
                )
                rdma.start()

        for s in range(N_DEV):
            @pl.when(s != my_pos)
            def _(s=s):
                recv = pltpu.make_async_remote_copy(
                    src_ref=gather_ref.at[pl.ds(s, 1), :],
                    dst_ref=gather_ref.at[pl.ds(s, 1), :],
                    send_sem=send_sems.at[s],
                    recv_sem=recv_sems.at[s],
                    device_id=(0,),
                    device_id_type=pl.DeviceIdType.MESH,
                )
                recv.wait_recv()

        out_ref[:, :] = jnp.sum(gather_ref[:, :], axis=0, keepdims=True)

        for q in range(N_DEV):
            @pl.when(q != my_pos)
            def _(q=q):
                snd = pltpu.make_async_remote_copy(
                    src_ref=gather_ref.at[pl.ds(my_pos, 1), :],
                    dst_ref=gather_ref.at[pl.ds(my_pos, 1), :],
                    send_sem=send_sems.at[q],
                    recv_sem=recv_sems.at[q],
                    device_id=(0,),
                    device_id_type=pl.DeviceIdType.MESH,
                )
                snd.wait_send()

    return pl.pallas_call(
        body,
        out_shape=jax.ShapeDtypeStruct((1, n), jnp.float32),
        in_specs=[pl.BlockSpec(memory_space=pltpu.VMEM)],
        out_specs=pl.BlockSpec(memory_space=pltpu.VMEM),
        scratch_shapes=[
            pltpu.VMEM((N_DEV, n), jnp.float32),
            pltpu.SemaphoreType.DMA((N_DEV,)),
            pltpu.SemaphoreType.DMA((N_DEV,)),
        ],
        compiler_params=pltpu.CompilerParams(collective_id=0),
    )(x)
